# baseline (speedup 1.0000x reference)
import functools

import jax
import jax.numpy as jnp
from jax import lax
from jax.experimental import pallas as pl
from jax.experimental.pallas import tpu as pltpu
from jax.experimental.pallas import tpu_sc as plsc

_BLK = 128
_LANES = 16
_PITCH = 136


def _pos_encoding(length: int, hidden: int) -> jax.Array:
    depth = hidden // 2
    positions = jnp.arange(length)[:, None].astype(jnp.float32)
    depths = jnp.arange(depth)[None, :].astype(jnp.float32) / depth
    angle_rates = 1.0 / (10000.0 ** depths)
    angle_rads = positions * angle_rates
    return jnp.concatenate(
        [jnp.sin(angle_rads), jnp.cos(angle_rads)], axis=-1
    ).astype(jnp.float32)


@functools.partial(jax.jit, static_argnames=("b_total", "hidden", "length"))
def _emb_lookup(xt2d, pos, table, *, b_total, hidden, length):
    info = plsc.get_sparse_core_info()
    nc, ns = info.num_cores, info.num_subcores
    nw = nc * ns
    n_btiles = b_total // _BLK
    per_w = (length * n_btiles) // nw
    scale = float(hidden) ** 0.5
    hgrp = hidden // 8
    nvr = hidden // _LANES

    mesh = plsc.VectorSubcoreMesh(core_axis_name="c", subcore_axis_name="s")

    @functools.partial(
        pl.kernel,
        mesh=mesh,
        compiler_params=pltpu.CompilerParams(
            use_tc_tiling_on_sc=False, needs_layout_passes=False),
        out_type=jax.ShapeDtypeStruct(
            (length, hgrp, n_btiles, 8, _BLK), jnp.float32),
        scratch_types=[
            pltpu.VMEM((per_w, _BLK), jnp.int32),
            pltpu.VMEM((_BLK, hidden), jnp.float32),
            pltpu.VMEM((_BLK, hidden), jnp.float32),
            pltpu.VMEM((length * hidden,), jnp.float32),
            pltpu.VMEM((hidden * _PITCH,), jnp.float32),
            pltpu.VMEM((hidden * _PITCH,), jnp.float32),
            pltpu.SemaphoreType.DMA,
            pltpu.SemaphoreType.DMA,
            pltpu.SemaphoreType.DMA,
            pltpu.SemaphoreType.DMA,
        ],
    )
    def k(x_hbm, pos_hbm, tab_hbm, out_hbm, idx_all, rows0, rows1,
          pos_v, ob0, ob1, sg0, sg1, so0, so1):
        wid = lax.axis_index("s") * nc + lax.axis_index("c")
        base_c = wid * per_w
        rows = (rows0, rows1)
        obuf = (ob0, ob1)
        sem_g = (sg0, sg1)
        sem_o = (so0, so1)

        pltpu.sync_copy(x_hbm.at[pl.ds(base_c, per_w)], idx_all)
        pltpu.sync_copy(pos_hbm, pos_v)

        iota = lax.iota(jnp.int32, _LANES)
        hvecs = [(iota + ci * _LANES) * _PITCH for ci in range(nvr)]

        def gather_copy(j, b):
            return pltpu.make_async_copy(
                tab_hbm.at[idx_all.at[j]], rows[b], sem_g[b])

        def store_copy(j, b, g1, s):
            c = base_c + j
            l = lax.shift_right_logical(c, 5)
            g0b = lax.rem(c, n_btiles)
            return pltpu.make_async_copy(
                obuf[b].at[pl.ds((g1 * 8 + s) * _PITCH, _BLK)],
                out_hbm.at[l, g1, g0b, s], sem_o[b])

        def step(j, b):
            @pl.when(j + 1 < per_w)
            def _():
                gather_copy(j + 1, 1 - b).start()

            gather_copy(j, b).wait()

            @pl.when(j >= 2)
            def _():
                for g1 in range(hgrp):
                    for s in range(8):
                        store_copy(j - 2, b, g1, s).wait()

            c = base_c + j
            l = lax.shift_right_logical(c, 5)
            rv = rows[b]
            ov = obuf[b]
            pvec = [pos_v[pl.ds(l * hidden + ci * _LANES, _LANES)]
                    for ci in range(nvr)]

            def row_body(r, carry):
                rsplat = jnp.zeros((_LANES,), jnp.int32) + r
                for ci in range(nvr):
                    v = rv[r, pl.ds(ci * _LANES, _LANES)]
                    plsc.store_scatter(
                        ov, [hvecs[ci] + rsplat], v * scale + pvec[ci])
                return carry

            lax.fori_loop(0, _BLK, row_body, 0, unroll=4)

            for g1 in range(hgrp):
                for s in range(8):
                    store_copy(j, b, g1, s).start()

        gather_copy(0, 0).start()

        def pair(i, carry):
            step(2 * i + 0, 0)
            step(2 * i + 1, 1)
            return carry

        lax.fori_loop(0, per_w // 2, pair, 0)
        for g1 in range(hgrp):
            for s in range(8):
                store_copy(per_w - 2, 0, g1, s).wait()
        for g1 in range(hgrp):
            for s in range(8):
                store_copy(per_w - 1, 1, g1, s).wait()

    return k(xt2d, pos, table)


def kernel(x, table):
    b_total, length = x.shape
    hidden = table.shape[1]
    pos = _pos_encoding(length, hidden).reshape(length * hidden)
    xt2d = x.T.reshape(length * b_total // _BLK, _BLK)
    out5 = _emb_lookup(
        xt2d, pos, table,
        b_total=b_total, hidden=hidden, length=length,
    )
    return (out5.transpose(2, 4, 0, 1, 3)
            .reshape(b_total, length, hidden))


# speedup vs baseline: 1.0523x; 1.0523x over previous
import functools

import jax
import jax.numpy as jnp
from jax import lax
from jax.experimental import pallas as pl
from jax.experimental.pallas import tpu as pltpu
from jax.experimental.pallas import tpu_sc as plsc

_BLK = 128
_LANES = 16
_PITCH = 136
_PADW = 128


def _pos_encoding(length: int, hidden: int) -> jax.Array:
    depth = hidden // 2
    positions = jnp.arange(length)[:, None].astype(jnp.float32)
    depths = jnp.arange(depth)[None, :].astype(jnp.float32) / depth
    angle_rates = 1.0 / (10000.0 ** depths)
    angle_rads = positions * angle_rates
    return jnp.concatenate(
        [jnp.sin(angle_rads), jnp.cos(angle_rads)], axis=-1
    ).astype(jnp.float32)


@functools.partial(jax.jit, static_argnames=("b_total", "hidden", "length"))
def _emb_lookup(xt2d, pos, tablep, *, b_total, hidden, length):
    info = plsc.get_sparse_core_info()
    nc, ns = info.num_cores, info.num_subcores
    nw = nc * ns
    n_btiles = b_total // _BLK
    per_w = (length * n_btiles) // nw
    scale = float(hidden) ** 0.5
    hgrp = hidden // 8
    nvr = hidden // _LANES

    mesh = plsc.VectorSubcoreMesh(core_axis_name="c", subcore_axis_name="s")

    @functools.partial(
        pl.kernel,
        mesh=mesh,
        compiler_params=pltpu.CompilerParams(
            use_tc_tiling_on_sc=False, needs_layout_passes=False),
        out_type=jax.ShapeDtypeStruct(
            (length, hgrp, n_btiles, 8, _BLK), jnp.float32),
        scratch_types=[
            pltpu.VMEM((per_w, _BLK), jnp.int32),
            pltpu.VMEM((_BLK, _PADW), jnp.float32),
            pltpu.VMEM((_BLK, _PADW), jnp.float32),
            pltpu.VMEM((length * hidden,), jnp.float32),
            pltpu.VMEM((hidden * _PITCH,), jnp.float32),
            pltpu.VMEM((hidden * _PITCH,), jnp.float32),
            pltpu.SemaphoreType.DMA,
            pltpu.SemaphoreType.DMA,
            pltpu.SemaphoreType.DMA,
            pltpu.SemaphoreType.DMA,
        ],
    )
    def k(x_hbm, pos_hbm, tab_hbm, out_hbm, idx_all, rows0, rows1,
          pos_v, ob0, ob1, sg0, sg1, so0, so1):
        wid = lax.axis_index("s") * nc + lax.axis_index("c")
        base_c = wid * per_w
        rows = (rows0, rows1)
        obuf = (ob0, ob1)
        sem_g = (sg0, sg1)
        sem_o = (so0, so1)

        pltpu.sync_copy(x_hbm.at[pl.ds(base_c, per_w)], idx_all)
        pltpu.sync_copy(pos_hbm, pos_v)

        iota = lax.iota(jnp.int32, _LANES)
        hvecs = [(iota + ci * _LANES) * _PITCH for ci in range(nvr)]

        def gather_copy(j, b):
            return pltpu.make_async_copy(
                tab_hbm.at[idx_all.at[j]], rows[b], sem_g[b])

        def store_copy(j, b, g1, s):
            c = base_c + j
            l = lax.shift_right_logical(c, 5)
            g0b = lax.rem(c, n_btiles)
            return pltpu.make_async_copy(
                obuf[b].at[pl.ds((g1 * 8 + s) * _PITCH, _BLK)],
                out_hbm.at[l, g1, g0b, s], sem_o[b])

        def step(j, b):
            @pl.when(j + 1 < per_w)
            def _():
                gather_copy(j + 1, 1 - b).start()

            gather_copy(j, b).wait()

            @pl.when(j >= 2)
            def _():
                for g1 in range(hgrp):
                    for s in range(8):
                        store_copy(j - 2, b, g1, s).wait()

            c = base_c + j
            l = lax.shift_right_logical(c, 5)
            rv = rows[b]
            ov = obuf[b]
            pvec = [pos_v[pl.ds(l * hidden + ci * _LANES, _LANES)]
                    for ci in range(nvr)]

            def row_body(r, carry):
                rsplat = jnp.zeros((_LANES,), jnp.int32) + r
                for ci in range(nvr):
                    v = rv[r, pl.ds(ci * _LANES, _LANES)]
                    plsc.store_scatter(
                        ov, [hvecs[ci] + rsplat], v * scale + pvec[ci])
                return carry

            lax.fori_loop(0, _BLK, row_body, 0, unroll=4)

            for g1 in range(hgrp):
                for s in range(8):
                    store_copy(j, b, g1, s).start()

        gather_copy(0, 0).start()

        def pair(i, carry):
            step(2 * i + 0, 0)
            step(2 * i + 1, 1)
            return carry

        lax.fori_loop(0, per_w // 2, pair, 0)
        for g1 in range(hgrp):
            for s in range(8):
                store_copy(per_w - 2, 0, g1, s).wait()
        for g1 in range(hgrp):
            for s in range(8):
                store_copy(per_w - 1, 1, g1, s).wait()

    return k(xt2d, pos, tablep)


def kernel(x, table):
    b_total, length = x.shape
    hidden = table.shape[1]
    pos = _pos_encoding(length, hidden).reshape(length * hidden)
    xt2d = x.T.reshape(length * b_total // _BLK, _BLK)
    tablep = jnp.pad(table, ((0, 0), (0, _PADW - hidden)))
    out5 = _emb_lookup(
        xt2d, pos, tablep,
        b_total=b_total, hidden=hidden, length=length,
    )
    return (out5.transpose(2, 4, 0, 1, 3)
            .reshape(b_total, length, hidden))


# speedup vs baseline: 1.0881x; 1.0340x over previous
import functools

import jax
import jax.numpy as jnp
from jax import lax
from jax.experimental import pallas as pl
from jax.experimental.pallas import tpu as pltpu
from jax.experimental.pallas import tpu_sc as plsc

_BLK = 128
_LANES = 16
_PITCH = 136
_PADW = 128


def _pos_encoding(length: int, hidden: int) -> jax.Array:
    depth = hidden // 2
    positions = jnp.arange(length)[:, None].astype(jnp.float32)
    depths = jnp.arange(depth)[None, :].astype(jnp.float32) / depth
    angle_rates = 1.0 / (10000.0 ** depths)
    angle_rads = positions * angle_rates
    return jnp.concatenate(
        [jnp.sin(angle_rads), jnp.cos(angle_rads)], axis=-1
    ).astype(jnp.float32)


@functools.partial(jax.jit, static_argnames=("b_total", "hidden", "length"))
def _emb_lookup(xt2d, pos, tablep, *, b_total, hidden, length):
    info = plsc.get_sparse_core_info()
    nc, ns = info.num_cores, info.num_subcores
    nw = nc * ns
    n_btiles = b_total // _BLK
    per_w = (length * n_btiles) // nw
    scale = float(hidden) ** 0.5
    hgrp = hidden // 8
    nvr = hidden // _LANES

    mesh = plsc.VectorSubcoreMesh(core_axis_name="c", subcore_axis_name="s")

    @functools.partial(
        pl.kernel,
        mesh=mesh,
        compiler_params=pltpu.CompilerParams(
            use_tc_tiling_on_sc=False, needs_layout_passes=False),
        out_type=jax.ShapeDtypeStruct(
            (length, hgrp, n_btiles, 8, _BLK), jnp.float32),
        scratch_types=[
            pltpu.VMEM((per_w, _BLK), jnp.int32),
            pltpu.VMEM((_BLK, _PADW), jnp.float32),
            pltpu.VMEM((_BLK, _PADW), jnp.float32),
            pltpu.VMEM((length * hidden,), jnp.float32),
            pltpu.VMEM((hidden * _PITCH,), jnp.float32),
            pltpu.VMEM((hidden * _PITCH,), jnp.float32),
            pltpu.SemaphoreType.DMA,
            pltpu.SemaphoreType.DMA,
            pltpu.SemaphoreType.DMA,
            pltpu.SemaphoreType.DMA,
        ],
    )
    def k(x_hbm, pos_hbm, tab_hbm, out_hbm, idx_all, rows0, rows1,
          pos_v, ob0, ob1, sg0, sg1, so0, so1):
        wid = lax.axis_index("s") * nc + lax.axis_index("c")
        base_c = wid * per_w
        rows = (rows0, rows1)
        obuf = (ob0, ob1)
        sem_g = (sg0, sg1)
        sem_o = (so0, so1)

        pltpu.sync_copy(x_hbm.at[pl.ds(base_c, per_w)], idx_all)
        pltpu.sync_copy(pos_hbm, pos_v)

        iota = lax.iota(jnp.int32, _LANES)
        hvecs = [(iota + ci * _LANES) * _PITCH for ci in range(nvr)]

        def gather_copy(j, b):
            return pltpu.make_async_copy(
                tab_hbm.at[idx_all.at[j]], rows[b], sem_g[b])

        def store_drain(b):
            return pltpu.make_async_copy(
                pos_hbm.at[pl.ds(0, 8 * _BLK * 8)],
                obuf[b].at[pl.ds(0, 8 * _BLK * 8)], sem_o[b])

        def store_copy(j, b, g1, s):
            c = base_c + j
            l = lax.shift_right_logical(c, 5)
            g0b = lax.rem(c, n_btiles)
            return pltpu.make_async_copy(
                obuf[b].at[pl.ds((g1 * 8 + s) * _PITCH, _BLK)],
                out_hbm.at[l, g1, g0b, s], sem_o[b])

        def step(j, b):
            @pl.when(j + 1 < per_w)
            def _():
                gather_copy(j + 1, 1 - b).start()

            gather_copy(j, b).wait()

            @pl.when(j >= 2)
            def _():
                store_drain(b).wait()

            c = base_c + j
            l = lax.shift_right_logical(c, 5)
            rv = rows[b]
            ov = obuf[b]
            pvec = [pos_v[pl.ds(l * hidden + ci * _LANES, _LANES)]
                    for ci in range(nvr)]

            def row_body(r, carry):
                rsplat = jnp.zeros((_LANES,), jnp.int32) + r
                for ci in range(nvr):
                    v = rv[r, pl.ds(ci * _LANES, _LANES)]
                    plsc.store_scatter(
                        ov, [hvecs[ci] + rsplat], v * scale + pvec[ci])
                return carry

            lax.fori_loop(0, _BLK, row_body, 0, unroll=8)

            for g1 in range(hgrp):
                for s in range(8):
                    store_copy(j, b, g1, s).start()

        gather_copy(0, 0).start()

        def pair(i, carry):
            step(2 * i + 0, 0)
            step(2 * i + 1, 1)
            return carry

        lax.fori_loop(0, per_w // 2, pair, 0)
        store_drain(0).wait()
        store_drain(1).wait()

    return k(xt2d, pos, tablep)


def kernel(x, table):
    b_total, length = x.shape
    hidden = table.shape[1]
    pos = _pos_encoding(length, hidden).reshape(length * hidden)
    xt2d = x.T.reshape(length * b_total // _BLK, _BLK)
    tablep = jnp.pad(table, ((0, 0), (0, _PADW - hidden)))
    out5 = _emb_lookup(
        xt2d, pos, tablep,
        b_total=b_total, hidden=hidden, length=length,
    )
    return (out5.transpose(2, 4, 0, 1, 3)
            .reshape(b_total, length, hidden))
